# Initial kernel scaffold; baseline (speedup 1.0000x reference)
import jax
import jax.numpy as jnp
from jax.experimental import pallas as pl


def kernel(x, segment_ids, W1, b1, W2, b2, W3, b3):
    raise NotImplementedError("write your pallas kernel here")


# speedup vs baseline: 5.3373x; 5.3373x over previous
import functools

import jax
import jax.numpy as jnp
from jax import lax
from jax.experimental import pallas as pl
from jax.experimental.pallas import tpu as pltpu
from jax.experimental.pallas import tpu_sc as plsc

N_ROWS = 320000
D = 128
S = 10000
CW = 16
CHUNK = 128
NC = 2
NS = 16
NW = NC * NS
NCHUNKS = N_ROWS // CHUNK
FULL_ITERS = NCHUNKS // NW
REM = NCHUNKS - FULL_ITERS * NW
FULL_HOPS = S // CHUNK
TAIL = S - FULL_HOPS * CHUNK
HOPS_PER_TILE = 5


def _sc_segment_sum(x, segment_ids, zrow, zcnt, ones):
    mesh = plsc.VectorSubcoreMesh(core_axis_name="c", subcore_axis_name="s",
                                  num_cores=NC)

    @functools.partial(
        pl.kernel,
        out_type=(
            jax.ShapeDtypeStruct((NC, S, D), jnp.float32),
            jax.ShapeDtypeStruct((NC, S, CW), jnp.float32),
        ),
        mesh=mesh,
        compiler_params=pltpu.CompilerParams(use_tc_tiling_on_sc=False),
        scratch_types=[
            pltpu.VMEM((CHUNK,), jnp.int32),
            pltpu.VMEM((CHUNK, D), jnp.float32),
            pltpu.VMEM((CHUNK, CW), jnp.float32),
            pltpu.VMEM((CHUNK, CW), jnp.float32),
            pltpu.VMEM_SHARED((S, D), jnp.float32),
            pltpu.VMEM_SHARED((S, CW), jnp.float32),
        ],
    )
    def k(x_hbm, ids_hbm, zrow_hbm, zcnt_hbm, ones_hbm,
          psum_hbm, pcnt_hbm,
          idx_v, rows_v, ones_v, cbuf_v, accum, caccum):
        cid = lax.axis_index("c")
        sid = lax.axis_index("s")
        wid = sid * NC + cid

        pltpu.sync_copy(zrow_hbm, rows_v)
        pltpu.sync_copy(zcnt_hbm, cbuf_v)

        def hops(fn):
            for j in range(HOPS_PER_TILE):
                h = sid + j * NS

                @pl.when(h < FULL_HOPS)
                def _():
                    fn(h * CHUNK, CHUNK)

            @pl.when(sid == NS - 1)
            def _():
                fn(FULL_HOPS * CHUNK, TAIL)

        def init_hop(off, n):
            pltpu.sync_copy(rows_v.at[pl.ds(0, n)],
                            accum.at[pl.ds(off, n)])
            pltpu.sync_copy(cbuf_v.at[pl.ds(0, n)],
                            caccum.at[pl.ds(off, n)])

        hops(init_hop)
        pltpu.sync_copy(ones_hbm, ones_v)
        plsc.subcore_barrier()

        def do_chunk(g):
            row0 = g * CHUNK
            pltpu.sync_copy(ids_hbm.at[pl.ds(row0, CHUNK)], idx_v)
            pltpu.sync_copy(x_hbm.at[pl.ds(row0, CHUNK)], rows_v)
            pltpu.sync_copy(rows_v, accum.at[idx_v], add=True)
            pltpu.sync_copy(ones_v, caccum.at[idx_v], add=True)

        def body(i, _):
            do_chunk(wid + i * NW)
            return 0

        lax.fori_loop(0, FULL_ITERS, body, 0)

        @pl.when(wid < REM)
        def _():
            do_chunk(FULL_ITERS * NW + wid)

        plsc.subcore_barrier()

        def export_hop(off, n):
            pltpu.sync_copy(accum.at[pl.ds(off, n)],
                            rows_v.at[pl.ds(0, n)])
            pltpu.sync_copy(rows_v.at[pl.ds(0, n)],
                            psum_hbm.at[cid, pl.ds(off, n)])
            pltpu.sync_copy(caccum.at[pl.ds(off, n)],
                            cbuf_v.at[pl.ds(0, n)])
            pltpu.sync_copy(cbuf_v.at[pl.ds(0, n)],
                            pcnt_hbm.at[cid, pl.ds(off, n)])

        hops(export_hop)

    return k(x, segment_ids, zrow, zcnt, ones)


def _tc_mlp_body(psum_ref, pcnt_ref, w1_ref, b1_ref, w2_ref, b2_ref,
                 w3_ref, b3_ref, out_ref):
    sums = psum_ref[0]
    cnt = pcnt_ref[0, :, 0:1]
    for c in range(1, NC):
        sums = sums + psum_ref[c]
        cnt = cnt + pcnt_ref[c, :, 0:1]
    pooled = sums / jnp.maximum(cnt, 1.0)
    h = jnp.maximum(jnp.dot(pooled, w1_ref[...],
                            preferred_element_type=jnp.float32)
                    + b1_ref[...], 0.0)
    h = jnp.maximum(jnp.dot(h, w2_ref[...],
                            preferred_element_type=jnp.float32)
                    + b2_ref[...], 0.0)
    out_ref[...] = (jnp.dot(h, w3_ref[...], preferred_element_type=jnp.float32)
                    + b3_ref[...])


def kernel(x, segment_ids, W1, b1, W2, b2, W3, b3):
    zrow = jnp.zeros((CHUNK, D), jnp.float32)
    zcnt = jnp.zeros((CHUNK, CW), jnp.float32)
    ones = jnp.ones((CHUNK, CW), jnp.float32)
    psum, pcnt = _sc_segment_sum(x, segment_ids, zrow, zcnt, ones)

    n_out = W3.shape[1]
    out = pl.pallas_call(
        _tc_mlp_body,
        out_shape=jax.ShapeDtypeStruct((S, n_out), jnp.float32),
    )(psum, pcnt, W1, b1.reshape(1, -1), W2, b2.reshape(1, -1),
      W3, b3.reshape(1, -1))
    return out


# speedup vs baseline: 8.4549x; 1.5841x over previous
import functools

import jax
import jax.numpy as jnp
from jax import lax
from jax.experimental import pallas as pl
from jax.experimental.pallas import tpu as pltpu
from jax.experimental.pallas import tpu_sc as plsc

N_ROWS = 320000
D = 128
S = 10000
CW = 16
CHUNK = 128
NC = 2
NS = 16
NW = NC * NS
NCHUNKS = N_ROWS // CHUNK
FULL_ITERS = NCHUNKS // NW
REM = NCHUNKS - FULL_ITERS * NW
FULL_HOPS = S // CHUNK
TAIL = S - FULL_HOPS * CHUNK
HOPS_PER_TILE = 5
NBUF = 2


def _sc_segment_sum(x, segment_ids, zrow, zcnt, ones):
    mesh = plsc.VectorSubcoreMesh(core_axis_name="c", subcore_axis_name="s",
                                  num_cores=NC)

    @functools.partial(
        pl.kernel,
        out_type=(
            jax.ShapeDtypeStruct((NC, S, D), jnp.float32),
            jax.ShapeDtypeStruct((NC, S, CW), jnp.float32),
        ),
        mesh=mesh,
        compiler_params=pltpu.CompilerParams(use_tc_tiling_on_sc=False),
        scratch_types=[
            pltpu.VMEM((NBUF, CHUNK), jnp.int32),
            pltpu.VMEM((NBUF, CHUNK, D), jnp.float32),
            pltpu.VMEM((CHUNK, CW), jnp.float32),
            pltpu.VMEM((CHUNK, CW), jnp.float32),
            pltpu.VMEM_SHARED((S, D), jnp.float32),
            pltpu.VMEM_SHARED((S, CW), jnp.float32),
            pltpu.SemaphoreType.DMA,
            pltpu.SemaphoreType.DMA,
            pltpu.SemaphoreType.DMA,
        ],
    )
    def k(x_hbm, ids_hbm, zrow_hbm, zcnt_hbm, ones_hbm,
          psum_hbm, pcnt_hbm,
          idx2, rows2, ones_v, cbuf_v, accum, caccum,
          sem0, sem1, sem_c):
        cid = lax.axis_index("c")
        sid = lax.axis_index("s")
        wid = sid * NC + cid
        sems = (sem0, sem1)

        pltpu.sync_copy(zrow_hbm, rows2.at[0])
        pltpu.sync_copy(zcnt_hbm, cbuf_v)

        def hops(fn):
            for j in range(HOPS_PER_TILE):
                h = sid + j * NS

                @pl.when(h < FULL_HOPS)
                def _():
                    fn(h * CHUNK, CHUNK)

            @pl.when(sid == NS - 1)
            def _():
                fn(FULL_HOPS * CHUNK, TAIL)

        def init_hop(off, n):
            pltpu.sync_copy(rows2.at[0, pl.ds(0, n)],
                            accum.at[pl.ds(off, n)])
            pltpu.sync_copy(cbuf_v.at[pl.ds(0, n)],
                            caccum.at[pl.ds(off, n)])

        hops(init_hop)
        pltpu.sync_copy(ones_hbm, ones_v)
        plsc.subcore_barrier()

        def load_descs(k_idx, b):
            row0 = (wid + k_idx * NW) * CHUNK
            return (
                pltpu.make_async_copy(ids_hbm.at[pl.ds(row0, CHUNK)],
                                      idx2.at[b], sems[b]),
                pltpu.make_async_copy(x_hbm.at[pl.ds(row0, CHUNK)],
                                      rows2.at[b], sems[b]),
            )

        def issue_load(k_idx, b):
            di, dr = load_descs(k_idx, b)
            di.start()
            dr.start()

        for b in range(NBUF):
            issue_load(b, b)

        def step(i, _):
            for b in range(NBUF):
                k_idx = i * NBUF + b
                di, dr = load_descs(k_idx, b)
                di.wait()
                dr.wait()
                cdesc = pltpu.async_copy(
                    ones_v, caccum.at[idx2.at[b]], sem_c, add=True)
                pltpu.sync_copy(rows2.at[b], accum.at[idx2.at[b]], add=True)
                cdesc.wait()

                @pl.when(k_idx + NBUF < FULL_ITERS)
                def _():
                    issue_load(k_idx + NBUF, b)
            return 0

        lax.fori_loop(0, FULL_ITERS // NBUF, step, 0)

        @pl.when(wid < REM)
        def _():
            row0 = (FULL_ITERS * NW + wid) * CHUNK
            pltpu.sync_copy(ids_hbm.at[pl.ds(row0, CHUNK)], idx2.at[0])
            pltpu.sync_copy(x_hbm.at[pl.ds(row0, CHUNK)], rows2.at[0])
            pltpu.sync_copy(rows2.at[0], accum.at[idx2.at[0]], add=True)
            pltpu.sync_copy(ones_v, caccum.at[idx2.at[0]], add=True)

        plsc.subcore_barrier()

        def export_hop(off, n):
            pltpu.sync_copy(accum.at[pl.ds(off, n)],
                            rows2.at[0, pl.ds(0, n)])
            pltpu.sync_copy(rows2.at[0, pl.ds(0, n)],
                            psum_hbm.at[cid, pl.ds(off, n)])
            pltpu.sync_copy(caccum.at[pl.ds(off, n)],
                            cbuf_v.at[pl.ds(0, n)])
            pltpu.sync_copy(cbuf_v.at[pl.ds(0, n)],
                            pcnt_hbm.at[cid, pl.ds(off, n)])

        hops(export_hop)

    return k(x, segment_ids, zrow, zcnt, ones)


def _tc_mlp_body(psum_ref, pcnt_ref, w1_ref, b1_ref, w2_ref, b2_ref,
                 w3_ref, b3_ref, out_ref):
    sums = psum_ref[0]
    cnt = pcnt_ref[0, :, 0:1]
    for c in range(1, NC):
        sums = sums + psum_ref[c]
        cnt = cnt + pcnt_ref[c, :, 0:1]
    pooled = sums / jnp.maximum(cnt, 1.0)
    h = jnp.maximum(jnp.dot(pooled, w1_ref[...],
                            preferred_element_type=jnp.float32)
                    + b1_ref[...], 0.0)
    h = jnp.maximum(jnp.dot(h, w2_ref[...],
                            preferred_element_type=jnp.float32)
                    + b2_ref[...], 0.0)
    out_ref[...] = (jnp.dot(h, w3_ref[...], preferred_element_type=jnp.float32)
                    + b3_ref[...])


def kernel(x, segment_ids, W1, b1, W2, b2, W3, b3):
    zrow = jnp.zeros((CHUNK, D), jnp.float32)
    zcnt = jnp.zeros((CHUNK, CW), jnp.float32)
    ones = jnp.ones((CHUNK, CW), jnp.float32)
    psum, pcnt = _sc_segment_sum(x, segment_ids, zrow, zcnt, ones)

    n_out = W3.shape[1]
    out = pl.pallas_call(
        _tc_mlp_body,
        out_shape=jax.ShapeDtypeStruct((S, n_out), jnp.float32),
    )(psum, pcnt, W1, b1.reshape(1, -1), W2, b2.reshape(1, -1),
      W3, b3.reshape(1, -1))
    return out


# speedup vs baseline: 8.7143x; 1.0307x over previous
import functools

import jax
import jax.numpy as jnp
from jax import lax
from jax.experimental import pallas as pl
from jax.experimental.pallas import tpu as pltpu
from jax.experimental.pallas import tpu_sc as plsc

N_ROWS = 320000
D = 128
S = 10000
CW = 8
CHUNK = 64
NC = 2
NS = 16
NW = NC * NS
NCHUNKS = N_ROWS // CHUNK
FULL_ITERS = NCHUNKS // NW
REM = NCHUNKS - FULL_ITERS * NW
FULL_HOPS = S // CHUNK
TAIL = S - FULL_HOPS * CHUNK
HOPS_PER_TILE = 10
NBUF = 4


def _sc_segment_sum(x, segment_ids, zrow, zcnt, ones):
    mesh = plsc.VectorSubcoreMesh(core_axis_name="c", subcore_axis_name="s",
                                  num_cores=NC)

    @functools.partial(
        pl.kernel,
        out_type=(
            jax.ShapeDtypeStruct((NC, S, D), jnp.float32),
            jax.ShapeDtypeStruct((NC, S, CW), jnp.float32),
        ),
        mesh=mesh,
        compiler_params=pltpu.CompilerParams(use_tc_tiling_on_sc=False),
        scratch_types=[
            pltpu.VMEM((NBUF, CHUNK), jnp.int32),
            pltpu.VMEM((NBUF, CHUNK, D), jnp.float32),
            pltpu.VMEM((CHUNK, CW), jnp.float32),
            pltpu.VMEM_SHARED((S, D), jnp.float32),
            pltpu.VMEM_SHARED((S, CW), jnp.float32),
        ] + [pltpu.SemaphoreType.DMA] * (3 * NBUF),
    )
    def k(x_hbm, ids_hbm, zrow_hbm, zcnt_hbm, ones_hbm,
          psum_hbm, pcnt_hbm,
          idx2, rows2, ones_v, accum, caccum, *sems):
        cid = lax.axis_index("c")
        sid = lax.axis_index("s")
        wid = sid * NC + cid
        sem_ld = sems[0:NBUF]
        sem_s = sems[NBUF:2 * NBUF]
        sem_c = sems[2 * NBUF:3 * NBUF]

        pltpu.sync_copy(zrow_hbm, rows2.at[0])
        pltpu.sync_copy(zcnt_hbm, ones_v)

        def hops(fn):
            for j in range(HOPS_PER_TILE):
                h = sid + j * NS

                @pl.when(h < FULL_HOPS)
                def _():
                    fn(h * CHUNK, CHUNK)

            @pl.when(sid == NS - 1)
            def _():
                fn(FULL_HOPS * CHUNK, TAIL)

        def init_hop(off, n):
            pltpu.sync_copy(rows2.at[0, pl.ds(0, n)],
                            accum.at[pl.ds(off, n)])
            pltpu.sync_copy(ones_v.at[pl.ds(0, n)],
                            caccum.at[pl.ds(off, n)])

        hops(init_hop)
        pltpu.sync_copy(ones_hbm, ones_v)
        plsc.subcore_barrier()

        def load_descs(k_idx, b):
            row0 = (wid + k_idx * NW) * CHUNK
            return (
                pltpu.make_async_copy(ids_hbm.at[pl.ds(row0, CHUNK)],
                                      idx2.at[b], sem_ld[b]),
                pltpu.make_async_copy(x_hbm.at[pl.ds(row0, CHUNK)],
                                      rows2.at[b], sem_ld[b]),
            )

        def issue_load(k_idx, b):
            di, dr = load_descs(k_idx, b)
            di.start()
            dr.start()

        def drain_scatter(b):
            pltpu.make_async_copy(rows2.at[b], accum.at[idx2.at[b]],
                                  sem_s[b]).wait()
            pltpu.make_async_copy(ones_v, caccum.at[idx2.at[b]],
                                  sem_c[b]).wait()

        for b in range(NBUF - 1):
            issue_load(b, b)

        def step(i, _):
            for b in range(NBUF):
                k_idx = i * NBUF + b
                kt = i * NBUF + b
                di, dr = load_descs(k_idx, b)
                di.wait()
                dr.wait()
                pltpu.async_copy(rows2.at[b], accum.at[idx2.at[b]],
                                 sem_s[b], add=True)
                pltpu.async_copy(ones_v, caccum.at[idx2.at[b]],
                                 sem_c[b], add=True)

                pb = (b - 1) % NBUF

                @pl.when(kt >= 1)
                def _():
                    drain_scatter(pb)

                nb = (b + NBUF - 1) % NBUF

                @pl.when(kt + NBUF - 1 < FULL_ITERS)
                def _():
                    issue_load(k_idx + NBUF - 1, nb)
            return 0

        lax.fori_loop(0, FULL_ITERS // NBUF, step, 0)
        drain_scatter((FULL_ITERS - 1) % NBUF)

        @pl.when(wid < REM)
        def _():
            row0 = (FULL_ITERS * NW + wid) * CHUNK
            pltpu.sync_copy(ids_hbm.at[pl.ds(row0, CHUNK)], idx2.at[0])
            pltpu.sync_copy(x_hbm.at[pl.ds(row0, CHUNK)], rows2.at[0])
            pltpu.sync_copy(rows2.at[0], accum.at[idx2.at[0]], add=True)
            pltpu.sync_copy(ones_v, caccum.at[idx2.at[0]], add=True)

        plsc.subcore_barrier()

        def export_hop(off, n):
            pltpu.sync_copy(accum.at[pl.ds(off, n)],
                            rows2.at[0, pl.ds(0, n)])
            pltpu.sync_copy(rows2.at[0, pl.ds(0, n)],
                            psum_hbm.at[cid, pl.ds(off, n)])
            pltpu.sync_copy(caccum.at[pl.ds(off, n)],
                            ones_v.at[pl.ds(0, n)])
            pltpu.sync_copy(ones_v.at[pl.ds(0, n)],
                            pcnt_hbm.at[cid, pl.ds(off, n)])

        hops(export_hop)

    return k(x, segment_ids, zrow, zcnt, ones)


def _tc_mlp_body(psum_ref, pcnt_ref, w1_ref, b1_ref, w2_ref, b2_ref,
                 w3_ref, b3_ref, out_ref):
    sums = psum_ref[0]
    cnt = pcnt_ref[0, :, 0:1]
    for c in range(1, NC):
        sums = sums + psum_ref[c]
        cnt = cnt + pcnt_ref[c, :, 0:1]
    pooled = sums / jnp.maximum(cnt, 1.0)
    h = jnp.maximum(jnp.dot(pooled, w1_ref[...],
                            preferred_element_type=jnp.float32)
                    + b1_ref[...], 0.0)
    h = jnp.maximum(jnp.dot(h, w2_ref[...],
                            preferred_element_type=jnp.float32)
                    + b2_ref[...], 0.0)
    out_ref[...] = (jnp.dot(h, w3_ref[...], preferred_element_type=jnp.float32)
                    + b3_ref[...])


def kernel(x, segment_ids, W1, b1, W2, b2, W3, b3):
    zrow = jnp.zeros((CHUNK, D), jnp.float32)
    zcnt = jnp.zeros((CHUNK, CW), jnp.float32)
    ones = jnp.ones((CHUNK, CW), jnp.float32)
    psum, pcnt = _sc_segment_sum(x, segment_ids, zrow, zcnt, ones)

    n_out = W3.shape[1]
    out = pl.pallas_call(
        _tc_mlp_body,
        out_shape=jax.ShapeDtypeStruct((S, n_out), jnp.float32),
    )(psum, pcnt, W1, b1.reshape(1, -1), W2, b2.reshape(1, -1),
      W3, b3.reshape(1, -1))
    return out


# speedup vs baseline: 9.0674x; 1.0405x over previous
import functools

import jax
import jax.numpy as jnp
from jax import lax
from jax.experimental import pallas as pl
from jax.experimental.pallas import tpu as pltpu
from jax.experimental.pallas import tpu_sc as plsc

N_ROWS = 320000
D = 128
S = 10000
CW = 8
CHUNK = 64
NC = 2
NS = 16
DH = D // NC
NCHUNKS = N_ROWS // CHUNK
FULL_ITERS = NCHUNKS // NS
REM = NCHUNKS - FULL_ITERS * NS
FULL_HOPS = S // CHUNK
TAIL = S - FULL_HOPS * CHUNK
HOPS_PER_TILE = 10
NBUF = 4


def _sc_segment_sum(x, segment_ids, zrow, zcnt, ones):
    mesh = plsc.VectorSubcoreMesh(core_axis_name="c", subcore_axis_name="s",
                                  num_cores=NC)

    @functools.partial(
        pl.kernel,
        out_type=(
            jax.ShapeDtypeStruct((S, D), jnp.float32),
            jax.ShapeDtypeStruct((S, CW), jnp.float32),
        ),
        mesh=mesh,
        compiler_params=pltpu.CompilerParams(use_tc_tiling_on_sc=False),
        scratch_types=[
            pltpu.VMEM((NBUF, CHUNK), jnp.int32),
            pltpu.VMEM((NBUF, CHUNK, DH), jnp.float32),
            pltpu.VMEM((CHUNK, CW), jnp.float32),
            pltpu.VMEM_SHARED((S, DH), jnp.float32),
            pltpu.VMEM_SHARED((S, CW), jnp.float32),
        ] + [pltpu.SemaphoreType.DMA] * (3 * NBUF),
    )
    def k(x_hbm, ids_hbm, zrow_hbm, zcnt_hbm, ones_hbm,
          psum_hbm, pcnt_hbm,
          idx2, rows2, ones_v, accum, caccum, *sems):
        cid = lax.axis_index("c")
        sid = lax.axis_index("s")
        col0 = cid * DH
        sem_ld = sems[0:NBUF]
        sem_s = sems[NBUF:2 * NBUF]
        sem_c = sems[2 * NBUF:3 * NBUF]

        pltpu.sync_copy(zrow_hbm, rows2.at[0])
        pltpu.sync_copy(zcnt_hbm, ones_v)

        def hops(fn):
            for j in range(HOPS_PER_TILE):
                h = sid + j * NS

                @pl.when(h < FULL_HOPS)
                def _():
                    fn(h * CHUNK, CHUNK)

            @pl.when(sid == NS - 1)
            def _():
                fn(FULL_HOPS * CHUNK, TAIL)

        def init_hop(off, n):
            pltpu.sync_copy(rows2.at[0, pl.ds(0, n)],
                            accum.at[pl.ds(off, n)])
            pltpu.sync_copy(ones_v.at[pl.ds(0, n)],
                            caccum.at[pl.ds(off, n)])

        hops(init_hop)
        pltpu.sync_copy(ones_hbm, ones_v)
        plsc.subcore_barrier()

        def load_descs(k_idx, b):
            row0 = (sid + k_idx * NS) * CHUNK
            return (
                pltpu.make_async_copy(ids_hbm.at[pl.ds(row0, CHUNK)],
                                      idx2.at[b], sem_ld[b]),
                pltpu.make_async_copy(
                    x_hbm.at[pl.ds(row0, CHUNK), pl.ds(col0, DH)],
                    rows2.at[b], sem_ld[b]),
            )

        def issue_load(k_idx, b):
            di, dr = load_descs(k_idx, b)
            di.start()
            dr.start()

        def scatter_chunk(b):
            pltpu.async_copy(rows2.at[b], accum.at[idx2.at[b]],
                             sem_s[b], add=True)

            @pl.when(cid == 0)
            def _():
                pltpu.async_copy(ones_v, caccum.at[idx2.at[b]],
                                 sem_c[b], add=True)

        def drain_scatter(b):
            pltpu.make_async_copy(rows2.at[b], accum.at[idx2.at[b]],
                                  sem_s[b]).wait()

            @pl.when(cid == 0)
            def _():
                pltpu.make_async_copy(ones_v, caccum.at[idx2.at[b]],
                                      sem_c[b]).wait()

        for b in range(NBUF - 1):
            issue_load(b, b)

        def step(i, _):
            for b in range(NBUF):
                k_idx = i * NBUF + b
                di, dr = load_descs(k_idx, b)
                di.wait()
                dr.wait()
                scatter_chunk(b)

                pb = (b - 1) % NBUF

                @pl.when(k_idx >= 1)
                def _():
                    drain_scatter(pb)

                @pl.when(k_idx + NBUF - 1 < FULL_ITERS)
                def _():
                    issue_load(k_idx + NBUF - 1, pb)
            return 0

        lax.fori_loop(0, FULL_ITERS // NBUF, step, 0)
        drain_scatter((FULL_ITERS - 1) % NBUF)

        @pl.when(sid < REM)
        def _():
            row0 = (FULL_ITERS * NS + sid) * CHUNK
            pltpu.sync_copy(ids_hbm.at[pl.ds(row0, CHUNK)], idx2.at[0])
            pltpu.sync_copy(x_hbm.at[pl.ds(row0, CHUNK), pl.ds(col0, DH)],
                            rows2.at[0])
            pltpu.sync_copy(rows2.at[0], accum.at[idx2.at[0]], add=True)

            @pl.when(cid == 0)
            def _():
                pltpu.sync_copy(ones_v, caccum.at[idx2.at[0]], add=True)

        plsc.subcore_barrier()

        def export_hop(off, n):
            pltpu.sync_copy(accum.at[pl.ds(off, n)],
                            rows2.at[0, pl.ds(0, n)])
            pltpu.sync_copy(rows2.at[0, pl.ds(0, n)],
                            psum_hbm.at[pl.ds(off, n), pl.ds(col0, DH)])

            @pl.when(cid == 0)
            def _():
                pltpu.sync_copy(caccum.at[pl.ds(off, n)],
                                ones_v.at[pl.ds(0, n)])
                pltpu.sync_copy(ones_v.at[pl.ds(0, n)],
                                pcnt_hbm.at[pl.ds(off, n)])

        hops(export_hop)

    return k(x, segment_ids, zrow, zcnt, ones)


def _tc_mlp_body(psum_ref, pcnt_ref, w1_ref, b1_ref, w2_ref, b2_ref,
                 w3_ref, b3_ref, out_ref):
    cnt = pcnt_ref[:, 0:1]
    pooled = psum_ref[...] / jnp.maximum(cnt, 1.0)
    h = jnp.maximum(jnp.dot(pooled, w1_ref[...],
                            preferred_element_type=jnp.float32)
                    + b1_ref[...], 0.0)
    h = jnp.maximum(jnp.dot(h, w2_ref[...],
                            preferred_element_type=jnp.float32)
                    + b2_ref[...], 0.0)
    out_ref[...] = (jnp.dot(h, w3_ref[...], preferred_element_type=jnp.float32)
                    + b3_ref[...])


def kernel(x, segment_ids, W1, b1, W2, b2, W3, b3):
    zrow = jnp.zeros((CHUNK, DH), jnp.float32)
    zcnt = jnp.zeros((CHUNK, CW), jnp.float32)
    ones = jnp.ones((CHUNK, CW), jnp.float32)
    psum, pcnt = _sc_segment_sum(x, segment_ids, zrow, zcnt, ones)

    n_out = W3.shape[1]
    out = pl.pallas_call(
        _tc_mlp_body,
        out_shape=jax.ShapeDtypeStruct((S, n_out), jnp.float32),
    )(psum, pcnt, W1, b1.reshape(1, -1), W2, b2.reshape(1, -1),
      W3, b3.reshape(1, -1))
    return out
